# baseline (speedup 1.0000x reference)
import functools

import jax
import jax.numpy as jnp
from jax import lax
from jax.experimental import pallas as pl
from jax.experimental.pallas import tpu as pltpu
from jax.experimental.pallas import tpu_sc as plsc

NC = 2
NS = 16
NW = NC * NS
LANES = 16
BSEG = 128
K = 1000


@functools.lru_cache(maxsize=None)
def _build_sc_call(n_nodes: int, n_edges: int):
    assert n_edges % NW == 0
    ew = n_edges // NW
    assert ew % K == 0
    nch = ew // K
    assert nch % 2 == 0 and nch >= 4
    nvec = K // LANES

    mesh = plsc.VectorSubcoreMesh(core_axis_name="c", subcore_axis_name="s")

    @functools.partial(
        pl.kernel,
        out_type=jax.ShapeDtypeStruct((NC, BSEG), jnp.float32),
        mesh=mesh,
        scratch_types=dict(
            sig_v=pltpu.VMEM((BSEG,), jnp.float32),
            srcb=[pltpu.VMEM((K,), jnp.int32)] * 2,
            dstb=[pltpu.VMEM((K,), jnp.int32)] * 2,
            batb=[pltpu.VMEM((K,), jnp.int32)] * 2,
            srows=[pltpu.VMEM((K, 16), jnp.float32)] * 2,
            drows=[pltpu.VMEM((K, 16), jnp.float32)] * 2,
            acc=pltpu.VMEM((BSEG,), jnp.float32),
            idx128=pltpu.VMEM((BSEG,), jnp.int32),
            shacc=pltpu.VMEM_SHARED((BSEG,), jnp.float32),
            isem=[pltpu.SemaphoreType.DMA] * 2,
            bsem=[pltpu.SemaphoreType.DMA] * 2,
            gsem_s=[pltpu.SemaphoreType.DMA] * 2,
            gsem_d=[pltpu.SemaphoreType.DMA] * 2,
        ),
        compiler_params=pltpu.CompilerParams(
            needs_layout_passes=False, use_tc_tiling_on_sc=False),
    )
    def sc_kernel(packed_hbm, sigf_hbm, src_hbm, dst_hbm, bat_hbm, out_hbm,
                  *, sig_v, srcb, dstb, batb, srows, drows, acc,
                  idx128, shacc, isem, bsem, gsem_s, gsem_d):
        cid = lax.axis_index("c")
        sid = lax.axis_index("s")
        wid = sid * NC + cid
        iota = lax.iota(jnp.int32, LANES)

        zero16 = jnp.zeros((LANES,), jnp.float32)
        for i in range(BSEG // LANES):
            acc[pl.ds(i * LANES, LANES)] = zero16
            idx128[pl.ds(i * LANES, LANES)] = iota + (i * LANES)

        pltpu.sync_copy(sigf_hbm, sig_v)

        @pl.when(sid == 0)
        def _zero_shared():
            pltpu.sync_copy(acc, shacc)

        plsc.subcore_barrier()

        col0 = jnp.zeros((LANES,), jnp.int32)
        col1 = jnp.full((LANES,), 1, jnp.int32)
        col2 = jnp.full((LANES,), 2, jnp.int32)
        col3 = jnp.full((LANES,), 3, jnp.int32)

        def issue_sd(c, b):
            base = wid * ew + c * K
            pltpu.async_copy(src_hbm.at[pl.ds(base, K)], srcb[b], isem[b])
            pltpu.async_copy(dst_hbm.at[pl.ds(base, K)], dstb[b], isem[b])

        def wait_sd(b):
            pltpu.make_async_copy(src_hbm.at[pl.ds(0, K)], srcb[b],
                                  isem[b]).wait()
            pltpu.make_async_copy(dst_hbm.at[pl.ds(0, K)], dstb[b],
                                  isem[b]).wait()

        def issue_bat(c, b):
            base = wid * ew + c * K
            pltpu.async_copy(bat_hbm.at[pl.ds(base, K)], batb[b], bsem[b])

        def wait_bat(b):
            pltpu.make_async_copy(bat_hbm.at[pl.ds(0, K)], batb[b],
                                  bsem[b]).wait()

        def issue_gather(b):
            pltpu.async_copy(packed_hbm.at[srcb[b]], srows[b], gsem_s[b])
            pltpu.async_copy(packed_hbm.at[dstb[b]], drows[b], gsem_d[b])

        def wait_gather(b):
            pltpu.make_async_copy(packed_hbm.at[srcb[b]], srows[b],
                                  gsem_s[b]).wait()
            pltpu.make_async_copy(packed_hbm.at[dstb[b]], drows[b],
                                  gsem_d[b]).wait()

        def compute(b):
            sr = srows[b]
            dr = drows[b]
            bb = batb[b]

            def vec_body(j, carry2):
                e0 = j * LANES
                ridx = e0 + iota
                xs = plsc.load_gather(sr, [ridx, col0])
                ys = plsc.load_gather(sr, [ridx, col1])
                zs = plsc.load_gather(sr, [ridx, col2])
                ts = plsc.load_gather(sr, [ridx, col3])
                xd = plsc.load_gather(dr, [ridx, col0])
                yd = plsc.load_gather(dr, [ridx, col1])
                zd = plsc.load_gather(dr, [ridx, col2])
                td = plsc.load_gather(dr, [ridx, col3])
                dx = xs - xd
                dy = ys - yd
                dz = zs - zd
                d2 = dx * dx + dy * dy + dz * dz
                sidx = (ts * 10.0 + td).astype(jnp.int32)
                sv = plsc.load_gather(sig_v, [sidx])
                rr = (sv * sv) / d2
                y = rr * rr * rr
                bv = bb[pl.ds(e0, LANES)]
                plsc.addupdate_scatter(acc, [bv], y)
                return carry2

            lax.fori_loop(0, nvec, vec_body, 0)

        for c in (0, 1):
            issue_sd(c, c)
            wait_sd(c)
            issue_gather(c)
            issue_bat(c, c)

        def pair_body(c2, carry):
            for b in (0, 1):
                c = c2 * 2 + b
                wait_gather(b)

                @pl.when(c + 2 < nch)
                def _prefetch_idx():
                    issue_sd(c + 2, b)

                wait_bat(b)
                compute(b)

                @pl.when(c + 2 < nch)
                def _next_gather():
                    wait_sd(b)
                    issue_gather(b)
                    issue_bat(c + 2, b)

            return carry

        lax.fori_loop(0, nch // 2, pair_body, 0)

        pltpu.sync_copy(acc, shacc.at[idx128], add=True)
        plsc.subcore_barrier()

        @pl.when(sid == 0)
        def _emit():
            pltpu.sync_copy(shacc, out_hbm.at[cid])

    return sc_kernel


def kernel(pos, sigma, atom_types, index_mapping, mapping_batch):
    n_nodes = pos.shape[0]
    n_edges = index_mapping.shape[1]
    packed = jnp.concatenate(
        [pos.astype(jnp.float32),
         atom_types.astype(jnp.float32)[:, None],
         jnp.zeros((n_nodes, 12), jnp.float32)], axis=1)
    sig_flat = jnp.pad(sigma.reshape(-1).astype(jnp.float32),
                       (0, BSEG - sigma.size))
    src = index_mapping[0].astype(jnp.int32)
    dst = index_mapping[1].astype(jnp.int32)
    bat = mapping_batch.astype(jnp.int32)
    part = _build_sc_call(n_nodes, n_edges)(packed, sig_flat, src, dst, bat)
    return part[0] + part[1]


# speedup vs baseline: 1.1164x; 1.1164x over previous
import functools

import jax
import jax.numpy as jnp
from jax import lax
from jax.experimental import pallas as pl
from jax.experimental.pallas import tpu as pltpu
from jax.experimental.pallas import tpu_sc as plsc

NC = 2
NS = 16
NW = NC * NS
LANES = 16
BSEG = 128
K = 1000


@functools.lru_cache(maxsize=None)
def _build_sc_call(n_nodes: int, n_edges: int):
    assert n_edges % NW == 0
    ew = n_edges // NW
    assert ew % K == 0
    nch = ew // K
    assert nch % 2 == 0 and nch >= 4
    nvec = K // LANES

    mesh = plsc.VectorSubcoreMesh(core_axis_name="c", subcore_axis_name="s")

    @functools.partial(
        pl.kernel,
        out_type=jax.ShapeDtypeStruct((NC, BSEG), jnp.float32),
        mesh=mesh,
        scratch_types=dict(
            sig_v=pltpu.VMEM((BSEG,), jnp.float32),
            srcb=[pltpu.VMEM((K,), jnp.int32)] * 2,
            dstb=[pltpu.VMEM((K,), jnp.int32)] * 2,
            batb=[pltpu.VMEM((K,), jnp.int32)] * 2,
            srows=[pltpu.VMEM((K, 8), jnp.float32)] * 2,
            drows=[pltpu.VMEM((K, 8), jnp.float32)] * 2,
            acc=pltpu.VMEM((BSEG,), jnp.float32),
            idx128=pltpu.VMEM((BSEG,), jnp.int32),
            shacc=pltpu.VMEM_SHARED((BSEG,), jnp.float32),
            isem=[pltpu.SemaphoreType.DMA] * 2,
            bsem=[pltpu.SemaphoreType.DMA] * 2,
            gsem_s=[pltpu.SemaphoreType.DMA] * 2,
            gsem_d=[pltpu.SemaphoreType.DMA] * 2,
        ),
        compiler_params=pltpu.CompilerParams(
            needs_layout_passes=False, use_tc_tiling_on_sc=False),
    )
    def sc_kernel(packed_hbm, sigf_hbm, src_hbm, dst_hbm, bat_hbm, out_hbm,
                  *, sig_v, srcb, dstb, batb, srows, drows, acc,
                  idx128, shacc, isem, bsem, gsem_s, gsem_d):
        cid = lax.axis_index("c")
        sid = lax.axis_index("s")
        wid = sid * NC + cid
        iota = lax.iota(jnp.int32, LANES)

        zero16 = jnp.zeros((LANES,), jnp.float32)
        for i in range(BSEG // LANES):
            acc[pl.ds(i * LANES, LANES)] = zero16
            idx128[pl.ds(i * LANES, LANES)] = iota + (i * LANES)

        pltpu.sync_copy(sigf_hbm, sig_v)

        @pl.when(sid == 0)
        def _zero_shared():
            pltpu.sync_copy(acc, shacc)

        plsc.subcore_barrier()

        col0 = jnp.zeros((LANES,), jnp.int32)
        col1 = jnp.full((LANES,), 1, jnp.int32)
        col2 = jnp.full((LANES,), 2, jnp.int32)
        col3 = jnp.full((LANES,), 3, jnp.int32)

        def issue_sd(c, b):
            base = wid * ew + c * K
            pltpu.async_copy(src_hbm.at[pl.ds(base, K)], srcb[b], isem[b])
            pltpu.async_copy(dst_hbm.at[pl.ds(base, K)], dstb[b], isem[b])

        def wait_sd(b):
            pltpu.make_async_copy(src_hbm.at[pl.ds(0, K)], srcb[b],
                                  isem[b]).wait()
            pltpu.make_async_copy(dst_hbm.at[pl.ds(0, K)], dstb[b],
                                  isem[b]).wait()

        def issue_bat(c, b):
            base = wid * ew + c * K
            pltpu.async_copy(bat_hbm.at[pl.ds(base, K)], batb[b], bsem[b])

        def wait_bat(b):
            pltpu.make_async_copy(bat_hbm.at[pl.ds(0, K)], batb[b],
                                  bsem[b]).wait()

        def issue_gather(b):
            pltpu.async_copy(packed_hbm.at[srcb[b]], srows[b], gsem_s[b])
            pltpu.async_copy(packed_hbm.at[dstb[b]], drows[b], gsem_d[b])

        def wait_gather(b):
            pltpu.make_async_copy(packed_hbm.at[srcb[b]], srows[b],
                                  gsem_s[b]).wait()
            pltpu.make_async_copy(packed_hbm.at[dstb[b]], drows[b],
                                  gsem_d[b]).wait()

        def compute(b):
            sr = srows[b]
            dr = drows[b]
            bb = batb[b]

            def vec_body(j, carry2):
                e0 = j * LANES
                ridx = e0 + iota
                xs = plsc.load_gather(sr, [ridx, col0])
                ys = plsc.load_gather(sr, [ridx, col1])
                zs = plsc.load_gather(sr, [ridx, col2])
                ts = plsc.load_gather(sr, [ridx, col3])
                xd = plsc.load_gather(dr, [ridx, col0])
                yd = plsc.load_gather(dr, [ridx, col1])
                zd = plsc.load_gather(dr, [ridx, col2])
                td = plsc.load_gather(dr, [ridx, col3])
                dx = xs - xd
                dy = ys - yd
                dz = zs - zd
                d2 = dx * dx + dy * dy + dz * dz
                sidx = (ts * 10.0 + td).astype(jnp.int32)
                sv = plsc.load_gather(sig_v, [sidx])
                rr = (sv * sv) / d2
                y = rr * rr * rr
                bv = bb[pl.ds(e0, LANES)]
                plsc.addupdate_scatter(acc, [bv], y)
                return carry2

            lax.fori_loop(0, nvec, vec_body, 0)

        for c in (0, 1):
            issue_sd(c, c)
            wait_sd(c)
            issue_gather(c)
            issue_bat(c, c)

        def pair_body(c2, carry):
            for b in (0, 1):
                c = c2 * 2 + b
                wait_gather(b)

                @pl.when(c + 2 < nch)
                def _prefetch_idx():
                    issue_sd(c + 2, b)

                wait_bat(b)
                compute(b)

                @pl.when(c + 2 < nch)
                def _next_gather():
                    wait_sd(b)
                    issue_gather(b)
                    issue_bat(c + 2, b)

            return carry

        lax.fori_loop(0, nch // 2, pair_body, 0)

        pltpu.sync_copy(acc, shacc.at[idx128], add=True)
        plsc.subcore_barrier()

        @pl.when(sid == 0)
        def _emit():
            pltpu.sync_copy(shacc, out_hbm.at[cid])

    return sc_kernel


def kernel(pos, sigma, atom_types, index_mapping, mapping_batch):
    n_nodes = pos.shape[0]
    n_edges = index_mapping.shape[1]
    packed = jnp.concatenate(
        [pos.astype(jnp.float32),
         atom_types.astype(jnp.float32)[:, None],
         jnp.zeros((n_nodes, 4), jnp.float32)], axis=1)
    sig_flat = jnp.pad(sigma.reshape(-1).astype(jnp.float32),
                       (0, BSEG - sigma.size))
    src = index_mapping[0].astype(jnp.int32)
    dst = index_mapping[1].astype(jnp.int32)
    bat = mapping_batch.astype(jnp.int32)
    part = _build_sc_call(n_nodes, n_edges)(packed, sig_flat, src, dst, bat)
    return part[0] + part[1]
